# Initial kernel scaffold; baseline (speedup 1.0000x reference)
import jax
import jax.numpy as jnp
from jax.experimental import pallas as pl


def kernel(x, adj, W, b):
    raise NotImplementedError("write your pallas kernel here")


# speedup vs baseline: 14.8634x; 14.8634x over previous
import functools

import jax
import jax.numpy as jnp
from jax import lax
from jax.experimental import pallas as pl
from jax.experimental.pallas import tpu as pltpu
from jax.experimental.pallas import tpu_sc as plsc

N = 10000
E = 320000
F = 128
C = 64
D = 128
NPAD = 10016
DUMMY = N

NC = 2
NS = 16
NW = NC * NS
EPW = E // NW
BE = 80
STEPS = EPW // BE
NROWS = 10240
RPT = NROWS // NS
ZR = 128

BM1 = 2504
BM2 = 2000


def _mm_body(x_ref, w_ref, o_ref):
    xw = lax.dot_general(x_ref[...], w_ref[...],
                         (((1,), (1,)), ((), ())),
                         preferred_element_type=jnp.float32)
    i = pl.program_id(0)
    rows = i * BM1 + lax.broadcasted_iota(jnp.int32, (BM1, 1), 0)
    ones = (rows < N).astype(jnp.float32)
    o_ref[...] = jnp.concatenate(
        [xw, ones, jnp.zeros((BM1, D - C - 1), jnp.float32)], axis=1)


def _sc_body(xw_hbm, row_hbm, col_hbm, part_hbm,
             rbuf, cbuf, abuf, gbuf, zbuf, acc, sem):
    cid = lax.axis_index("c")
    tid = lax.axis_index("s")
    wid = cid * NS + tid
    r0 = tid * RPT

    def zfill(i, carry):
        for j in range(D // 16):
            zbuf[i, pl.ds(j * 16, 16)] = jnp.zeros((16,), jnp.float32)
        return carry
    lax.fori_loop(0, ZR, zfill, 0)
    for k in range(RPT // ZR):
        pltpu.sync_copy(zbuf, acc.at[pl.ds(r0 + k * ZR, ZR)])
    plsc.subcore_barrier()

    ebase = wid * EPW

    def step(s, carry):
        b = ebase + s * BE
        pltpu.sync_copy(row_hbm.at[pl.ds(b, BE)], rbuf)
        pltpu.sync_copy(col_hbm.at[pl.ds(b, BE)], cbuf)
        for j in range(BE // 16):
            sl = pl.ds(j * 16, 16)
            r = rbuf[sl]
            c = cbuf[sl]
            abuf[sl] = jnp.where(r == c, jnp.full((16,), DUMMY, jnp.int32), r)
        pltpu.async_copy(xw_hbm.at[abuf], gbuf, sem).wait()
        pltpu.sync_copy(gbuf, acc.at[cbuf], add=True)
        return carry
    lax.fori_loop(0, STEPS, step, 0)

    plsc.subcore_barrier()
    pltpu.sync_copy(acc.at[pl.ds(r0, RPT)],
                    part_hbm.at[cid, pl.ds(r0, RPT)])


def _combine_body(p_ref, xw_ref, b_ref, o_ref):
    p = p_ref[0] + p_ref[1]
    num = p[:, :C] + xw_ref[:, :C]
    deg = p[:, C:C + 1] + 1.0
    o_ref[...] = num / deg + b_ref[...]


def kernel(x, adj, W, b):
    xp = jnp.pad(x, ((0, NPAD - N), (0, 0)))
    row = adj[0]
    col = adj[1]

    xw_aug = pl.pallas_call(
        _mm_body,
        grid=(NPAD // BM1,),
        in_specs=[
            pl.BlockSpec((BM1, F), lambda i: (i, 0)),
            pl.BlockSpec((C, F), lambda i: (0, 0)),
        ],
        out_specs=pl.BlockSpec((BM1, D), lambda i: (i, 0)),
        out_shape=jax.ShapeDtypeStruct((NPAD, D), jnp.float32),
    )(xp, W)

    mesh = plsc.VectorSubcoreMesh(core_axis_name="c", subcore_axis_name="s")
    part = pl.kernel(
        _sc_body,
        out_type=jax.ShapeDtypeStruct((NC, NROWS, D), jnp.float32),
        mesh=mesh,
        scratch_types=[
            pltpu.VMEM((BE,), jnp.int32),
            pltpu.VMEM((BE,), jnp.int32),
            pltpu.VMEM((BE,), jnp.int32),
            pltpu.VMEM((BE, D), jnp.float32),
            pltpu.VMEM((ZR, D), jnp.float32),
            pltpu.VMEM_SHARED((NROWS, D), jnp.float32),
            pltpu.SemaphoreType.DMA,
        ],
    )(xw_aug, row, col)

    out = pl.pallas_call(
        _combine_body,
        grid=(N // BM2,),
        in_specs=[
            pl.BlockSpec((NC, BM2, D), lambda i: (0, i, 0)),
            pl.BlockSpec((BM2, D), lambda i: (i, 0)),
            pl.BlockSpec((1, C), lambda i: (0, 0)),
        ],
        out_specs=pl.BlockSpec((BM2, C), lambda i: (i, 0)),
        out_shape=jax.ShapeDtypeStruct((N, C), jnp.float32),
    )(part, xw_aug, b.reshape(1, C))

    return out


# speedup vs baseline: 31.2430x; 2.1020x over previous
import functools

import jax
import jax.numpy as jnp
from jax import lax
from jax.experimental import pallas as pl
from jax.experimental.pallas import tpu as pltpu
from jax.experimental.pallas import tpu_sc as plsc

N = 10000
E = 320000
F = 128
C = 64
D = 128
NPAD = 10016
DUMMY = N

NC = 2
NS = 16
NW = NC * NS
EPW = E // NW
BE = 80
STEPS = EPW // BE
NROWS = 10240
RPT = NROWS // NS
ZR = 32

BM1 = 2504
BM2 = 2000


def _mm_body(x_ref, w_ref, o_ref):
    xw = lax.dot_general(x_ref[...], w_ref[...],
                         (((1,), (1,)), ((), ())),
                         preferred_element_type=jnp.float32)
    i = pl.program_id(0)
    rows = i * BM1 + lax.broadcasted_iota(jnp.int32, (BM1, 1), 0)
    ones = (rows < N).astype(jnp.float32)
    o_ref[...] = jnp.concatenate(
        [xw, ones, jnp.zeros((BM1, D - C - 1), jnp.float32)], axis=1)


def _sc_body(xw_hbm, row_hbm, col_hbm, part_hbm,
             rfull, cfull, ab0, cb0, gb0, ab1, cb1, gb1, zbuf, acc, sg0, sg1):
    cid = lax.axis_index("c")
    tid = lax.axis_index("s")
    wid = cid * NS + tid
    r0 = tid * RPT
    ebase = wid * EPW

    pltpu.sync_copy(row_hbm.at[pl.ds(ebase, EPW)], rfull)
    pltpu.sync_copy(col_hbm.at[pl.ds(ebase, EPW)], cfull)

    def zfill(i, carry):
        for j in range(D // 16):
            zbuf[i, pl.ds(j * 16, 16)] = jnp.zeros((16,), jnp.float32)
        return carry
    lax.fori_loop(0, ZR, zfill, 0)
    for k in range(RPT // ZR):
        pltpu.sync_copy(zbuf, acc.at[pl.ds(r0 + k * ZR, ZR)])
    plsc.subcore_barrier()

    bufs = ((ab0, cb0, gb0, sg0), (ab1, cb1, gb1, sg1))

    def stage(s, p):
        ab, cb, gb, sg = bufs[p]
        off = s * BE
        for j in range(BE // 16):
            sl = pl.ds(off + j * 16, 16)
            dl = pl.ds(j * 16, 16)
            r = rfull[sl]
            c = cfull[sl]
            ab[dl] = jnp.where(r == c, jnp.full((16,), DUMMY, jnp.int32), r)
            cb[dl] = c
        pltpu.async_copy(xw_hbm.at[ab], gb, sg)

    def drain(p):
        ab, cb, gb, sg = bufs[p]
        pltpu.make_async_copy(xw_hbm.at[ab], gb, sg).wait()
        pltpu.sync_copy(gb, acc.at[cb], add=True)

    stage(0, 0)

    def body(i, carry):
        stage(2 * i + 1, 1)
        drain(0)
        stage(2 * i + 2, 0)
        drain(1)
        return carry
    lax.fori_loop(0, (STEPS - 1) // 2, body, 0)
    drain(0)

    plsc.subcore_barrier()
    pltpu.sync_copy(acc.at[pl.ds(r0, RPT)],
                    part_hbm.at[cid, pl.ds(r0, RPT)])


def _combine_body(p_ref, xw_ref, b_ref, o_ref):
    p = p_ref[0] + p_ref[1]
    num = p[:, :C] + xw_ref[:, :C]
    deg = p[:, C:C + 1] + 1.0
    o_ref[...] = num / deg + b_ref[...]


def kernel(x, adj, W, b):
    xp = jnp.pad(x, ((0, NPAD - N), (0, 0)))
    row = adj[0]
    col = adj[1]

    xw_aug = pl.pallas_call(
        _mm_body,
        grid=(NPAD // BM1,),
        in_specs=[
            pl.BlockSpec((BM1, F), lambda i: (i, 0)),
            pl.BlockSpec((C, F), lambda i: (0, 0)),
        ],
        out_specs=pl.BlockSpec((BM1, D), lambda i: (i, 0)),
        out_shape=jax.ShapeDtypeStruct((NPAD, D), jnp.float32),
    )(xp, W)

    mesh = plsc.VectorSubcoreMesh(core_axis_name="c", subcore_axis_name="s")
    part = pl.kernel(
        _sc_body,
        out_type=jax.ShapeDtypeStruct((NC, NROWS, D), jnp.float32),
        mesh=mesh,
        scratch_types=[
            pltpu.VMEM((EPW,), jnp.int32),
            pltpu.VMEM((EPW,), jnp.int32),
            pltpu.VMEM((BE,), jnp.int32),
            pltpu.VMEM((BE,), jnp.int32),
            pltpu.VMEM((BE, D), jnp.float32),
            pltpu.VMEM((BE,), jnp.int32),
            pltpu.VMEM((BE,), jnp.int32),
            pltpu.VMEM((BE, D), jnp.float32),
            pltpu.VMEM((ZR, D), jnp.float32),
            pltpu.VMEM_SHARED((NROWS, D), jnp.float32),
            pltpu.SemaphoreType.DMA,
            pltpu.SemaphoreType.DMA,
        ],
    )(xw_aug, row, col)

    out = pl.pallas_call(
        _combine_body,
        grid=(N // BM2,),
        in_specs=[
            pl.BlockSpec((NC, BM2, D), lambda i: (0, i, 0)),
            pl.BlockSpec((BM2, D), lambda i: (i, 0)),
            pl.BlockSpec((1, C), lambda i: (0, 0)),
        ],
        out_specs=pl.BlockSpec((BM2, C), lambda i: (i, 0)),
        out_shape=jax.ShapeDtypeStruct((N, C), jnp.float32),
    )(part, xw_aug, b.reshape(1, C))

    return out
